# Initial kernel scaffold; baseline (speedup 1.0000x reference)
import jax
import jax.numpy as jnp
from jax.experimental import pallas as pl


def kernel(y, m, sd, p):
    raise NotImplementedError("write your pallas kernel here")


# speedup vs baseline: 8.4866x; 8.4866x over previous
import jax
import jax.numpy as jnp
from jax.experimental import pallas as pl

_B, _K, _C, _T = 256, 1024, 32, 8
_D = _C * _T


def _vq_body(y_ref, m_ref, sd_ref, p_ref, z_ref, mo_ref, sdo_ref, po_ref):
    yf = y_ref[:]
    mf = m_ref[:]

    g = jax.lax.dot_general(yf, mf, (((1,), (1,)), ((), ())),
                            precision=jax.lax.Precision.HIGHEST)
    m2 = jax.lax.dot_general(jnp.ones((1, _D), jnp.float32), mf * mf,
                             (((1,), (1,)), ((), ())),
                             precision=jax.lax.Precision.HIGHEST)
    d2 = m2 - 2.0 * g

    kiota = jax.lax.broadcasted_iota(jnp.int32, (_B, _K), 1)
    biota = jax.lax.broadcasted_iota(jnp.int32, (_B, _K), 0)

    dmin = jnp.min(d2, axis=1, keepdims=True)
    z2 = jnp.min(jnp.where(d2 == dmin, kiota, _K), axis=1,
                 keepdims=True)
    z_ref[:] = z2

    onehot = z2 == kiota
    iwin = jnp.max(jnp.where(onehot, biota, -1), axis=0,
                   keepdims=True)
    count_row = jnp.sum(onehot.astype(jnp.float32), axis=0,
                        keepdims=True)
    po_ref[:] = p_ref[:] + count_row

    win = ((biota == iwin) & (iwin >= 0)).astype(jnp.float32)
    ywin = jax.lax.dot_general(win, yf, (((0,), (0,)), ((), ())),
                               precision=jax.lax.Precision.HIGHEST)
    count_col = jax.lax.dot_general(
        onehot.astype(jnp.float32), jnp.ones((_B, 1), jnp.float32),
        (((0,), (0,)), ((), ())),
        precision=jax.lax.Precision.HIGHEST)
    assigned = count_col > 0.0

    mn = mf * 0.01 + ywin * 0.99
    mo_ref[:] = jnp.where(assigned, mn, mf)
    dlt = mn - ywin
    sdf = sd_ref[:]
    sdo_ref[:] = jnp.where(assigned, dlt * dlt * 0.01 + sdf * 0.99, sdf)


def kernel(y, m, sd, p):
    yf = y.reshape(_B, _D)
    mf = m.reshape(_K, _D)
    sdf = sd.reshape(_K, _D)
    p2 = p.reshape(1, _K)
    z2, mo, sdo, po = pl.pallas_call(
        _vq_body,
        out_shape=(
            jax.ShapeDtypeStruct((_B, 1), jnp.int32),
            jax.ShapeDtypeStruct((_K, _D), jnp.float32),
            jax.ShapeDtypeStruct((_K, _D), jnp.float32),
            jax.ShapeDtypeStruct((1, _K), jnp.float32),
        ),
    )(yf, mf, sdf, p2)
    return (z2.reshape(_B), mo.reshape(_K, _C, _T),
            sdo.reshape(_K, _C, _T), po.reshape(_K))
